# baseline (speedup 1.0000x reference)
import functools

import jax
import jax.numpy as jnp
from jax import lax
from jax.experimental import pallas as pl
from jax.experimental.pallas import tpu as pltpu
from jax.experimental.pallas import tpu_sc as plsc

N = 10000
E = 320000
D = 128
NC = 2
NS = 16
NW = NC * NS
B = 128
BPT = 80
PAIRS = BPT // 2
EPAD = NW * BPT * B
ROWS2D = EPAD // B
DUMMY = N
ACCN = N + 8
RPT = 624
TAIL = N - NS * RPT
ZR = 8

_mesh = plsc.VectorSubcoreMesh(core_axis_name="c", subcore_axis_name="s")


def _zero_shared(acc_sh, zrows_v, s, width_chunks):
    for r in range(ZR):
        for kk in range(width_chunks):
            zrows_v[r, pl.ds(16 * kk, 16)] = jnp.zeros((16,), jnp.float32)

    def zcopy(k, carry):
        pltpu.sync_copy(zrows_v, acc_sh.at[pl.ds(s * RPT + k * ZR, ZR)])
        return carry
    lax.fori_loop(0, RPT // ZR, zcopy, 0)

    @pl.when(s == NS - 1)
    def _():
        pltpu.sync_copy(zrows_v, acc_sh.at[pl.ds(NS * RPT, ZR)])
        pltpu.sync_copy(zrows_v, acc_sh.at[pl.ds(NS * RPT + ZR, ZR)])


@functools.partial(
    pl.kernel,
    out_type=jax.ShapeDtypeStruct((NC, N, 16), jnp.float32),
    mesh=_mesh,
    scratch_types=[
        pltpu.VMEM((2, B), jnp.int32),
        pltpu.VMEM((B, 16), jnp.float32),
        pltpu.VMEM((ZR, 16), jnp.float32),
        pltpu.VMEM_SHARED((ACCN, 16), jnp.float32),
    ],
)
def _deg_kernel(dst_hbm, out_hbm, didx_v, ones_v, zrows_v, acc_sh):
    c = lax.axis_index("c")
    s = lax.axis_index("s")
    wid = c * NS + s
    ebase = wid * BPT * B

    def ofill(r, carry):
        ones_v[r] = jnp.ones((16,), jnp.float32)
        return carry
    lax.fori_loop(0, B, ofill, 0)

    _zero_shared(acc_sh, zrows_v, s, 1)
    plsc.subcore_barrier()

    def batch(i, carry):
        pltpu.sync_copy(dst_hbm.at[pl.ds(ebase + i * B, B)], didx_v.at[0])
        pltpu.sync_copy(ones_v, acc_sh.at[didx_v.at[0]], add=True)
        return carry
    lax.fori_loop(0, BPT, batch, 0)

    plsc.subcore_barrier()
    _copy_out(acc_sh, out_hbm, c, s)


def _copy_out(acc_sh, out_hbm, c, s):
    pltpu.sync_copy(acc_sh.at[pl.ds(s * RPT, RPT)],
                    out_hbm.at[c, pl.ds(s * RPT, RPT)])

    @pl.when(s == NS - 1)
    def _():
        pltpu.sync_copy(acc_sh.at[pl.ds(NS * RPT, TAIL)],
                        out_hbm.at[c, pl.ds(NS * RPT, TAIL)])


@functools.partial(
    pl.kernel,
    out_type=jax.ShapeDtypeStruct((NC, N, D), jnp.float32),
    mesh=_mesh,
    scratch_types=[
        pltpu.VMEM((2, B), jnp.int32),
        pltpu.VMEM((2, B), jnp.int32),
        pltpu.VMEM((B, D), jnp.float32),
        pltpu.VMEM((B, D), jnp.float32),
        pltpu.VMEM((ZR, D), jnp.float32),
        pltpu.VMEM_SHARED((ACCN, D), jnp.float32),
        pltpu.SemaphoreType.DMA,
        pltpu.SemaphoreType.DMA,
    ],
)
def _agg_kernel(hp_hbm, src_hbm, dst_hbm, out_hbm,
                sidx_v, didx_v, r0_v, r1_v, zrows_v, acc_sh, sem0, sem1):
    c = lax.axis_index("c")
    s = lax.axis_index("s")
    wid = c * NS + s
    ebase = wid * BPT * B

    _zero_shared(acc_sh, zrows_v, s, D // 16)
    plsc.subcore_barrier()

    pltpu.sync_copy(src_hbm.at[pl.ds(ebase, B)], sidx_v.at[0])
    pltpu.sync_copy(dst_hbm.at[pl.ds(ebase, B)], didx_v.at[0])
    pltpu.async_copy(hp_hbm.at[sidx_v.at[0]], r0_v, sem0)

    def pair(k, carry):
        o1 = ebase + (2 * k + 1) * B
        pltpu.sync_copy(src_hbm.at[pl.ds(o1, B)], sidx_v.at[1])
        pltpu.sync_copy(dst_hbm.at[pl.ds(o1, B)], didx_v.at[1])
        pltpu.async_copy(hp_hbm.at[sidx_v.at[1]], r1_v, sem1)
        pltpu.make_async_copy(hp_hbm.at[sidx_v.at[0]], r0_v, sem0).wait()
        pltpu.sync_copy(r0_v, acc_sh.at[didx_v.at[0]], add=True)
        @pl.when(k < PAIRS - 1)
        def _():
            o2 = ebase + (2 * k + 2) * B
            pltpu.sync_copy(src_hbm.at[pl.ds(o2, B)], sidx_v.at[0])
            pltpu.sync_copy(dst_hbm.at[pl.ds(o2, B)], didx_v.at[0])
            pltpu.async_copy(hp_hbm.at[sidx_v.at[0]], r0_v, sem0)
        pltpu.make_async_copy(hp_hbm.at[sidx_v.at[1]], r1_v, sem1).wait()
        pltpu.sync_copy(r1_v, acc_sh.at[didx_v.at[1]], add=True)
        return carry
    lax.fori_loop(0, PAIRS, pair, 0)

    plsc.subcore_barrier()
    _copy_out(acc_sh, out_hbm, c, s)


ROWS_TC = 1000
GRID_TC = N // ROWS_TC


def _dis_block(p_ref):
    p0 = p_ref[0, :, 0:1]
    p1 = p_ref[1, :, 0:1]
    return lax.rsqrt(p0 + p1 + 1.0)


def _tc1_body(x_ref, w_ref, p_ref, o_ref):
    dis = _dis_block(p_ref)
    o_ref[...] = jnp.dot(x_ref[...], w_ref[...],
                         preferred_element_type=jnp.float32) * dis


def _tc2_body(a_ref, hp_ref, p_ref, b_ref, w_ref, o_ref):
    dis = _dis_block(p_ref)
    h = jnp.maximum((a_ref[0] + a_ref[1] + hp_ref[...]) * dis + b_ref[...],
                    0.0)
    o_ref[...] = jnp.dot(h, w_ref[...],
                         preferred_element_type=jnp.float32) * dis


def _tc3_body(a_ref, hp_ref, p_ref, b_ref, o_ref):
    dis = _dis_block(p_ref)
    o_ref[...] = (a_ref[0] + a_ref[1] + hp_ref[...]) * dis + b_ref[...]


_row_spec = pl.BlockSpec((ROWS_TC, D), lambda i: (i, 0))
_acc_spec = pl.BlockSpec((NC, ROWS_TC, D), lambda i: (0, i, 0))
_p_spec = pl.BlockSpec((NC, ROWS_TC, 16), lambda i: (0, i, 0))
_w_spec = pl.BlockSpec((D, D), lambda i: (0, 0))
_b_spec = pl.BlockSpec((1, D), lambda i: (0, 0))
_out_sds = jax.ShapeDtypeStruct((N, D), jnp.float32)

_tc1 = pl.pallas_call(
    _tc1_body, grid=(GRID_TC,),
    in_specs=[_row_spec, _w_spec, _p_spec],
    out_specs=_row_spec, out_shape=_out_sds)

_tc2 = pl.pallas_call(
    _tc2_body, grid=(GRID_TC,),
    in_specs=[_acc_spec, _row_spec, _p_spec, _b_spec, _w_spec],
    out_specs=_row_spec, out_shape=_out_sds)

_tc3 = pl.pallas_call(
    _tc3_body, grid=(GRID_TC,),
    in_specs=[_acc_spec, _row_spec, _p_spec, _b_spec],
    out_specs=_row_spec, out_shape=_out_sds)


def kernel(x, edge_index, W1, b1, W2, b2):
    src = edge_index[0].astype(jnp.int32)
    dst = edge_index[1].astype(jnp.int32)
    pad = EPAD - E
    srcp = jnp.concatenate([src, jnp.zeros((pad,), jnp.int32)])
    dstp = jnp.concatenate([dst, jnp.full((pad,), DUMMY, jnp.int32)])
    b1r = b1.reshape(1, D)
    b2r = b2.reshape(1, D)

    degp = _deg_kernel(dstp)
    h1p = _tc1(x, W1, degp)
    a1 = _agg_kernel(h1p, srcp, dstp)
    h2p = _tc2(a1, h1p, degp, b1r, W2)
    a2 = _agg_kernel(h2p, srcp, dstp)
    out = _tc3(a2, h2p, degp, b2r)
    return out


# speedup vs baseline: 2.3684x; 2.3684x over previous
import functools

import jax
import jax.numpy as jnp
from jax import lax
from jax.experimental import pallas as pl
from jax.experimental.pallas import tpu as pltpu
from jax.experimental.pallas import tpu_sc as plsc

N = 10000
E = 320000
D = 128
NC = 2
NS = 16
NW = NC * NS
BD = 128
BPTD = 80
DPAIRS = BPTD // 2
EPAD = NW * BPTD * BD
B = 80
BPT = 125
PAIRS = BPT // 2
DUMMY = N
ACCN = N + 8
RPT = 624
TAIL = N - NS * RPT
ZR = 24

_mesh = plsc.VectorSubcoreMesh(core_axis_name="c", subcore_axis_name="s")


def _zero_shared(acc_sh, zrows_v, s, width_chunks, semz):
    for r in range(ZR):
        for kk in range(width_chunks):
            zrows_v[r, pl.ds(16 * kk, 16)] = jnp.zeros((16,), jnp.float32)

    def zfire(k, carry):
        pltpu.async_copy(zrows_v, acc_sh.at[pl.ds(s * RPT + k * ZR, ZR)],
                         semz)
        return carry
    lax.fori_loop(0, RPT // ZR, zfire, 0)

    @pl.when(s == NS - 1)
    def _():
        pltpu.async_copy(zrows_v, acc_sh.at[pl.ds(NS * RPT, ZR)], semz)

    def zdrain(k, carry):
        pltpu.make_async_copy(
            zrows_v, acc_sh.at[pl.ds(s * RPT + k * ZR, ZR)], semz).wait()
        return carry
    lax.fori_loop(0, RPT // ZR, zdrain, 0)

    @pl.when(s == NS - 1)
    def _():
        pltpu.make_async_copy(
            zrows_v, acc_sh.at[pl.ds(NS * RPT, ZR)], semz).wait()


@functools.partial(
    pl.kernel,
    out_type=jax.ShapeDtypeStruct((NC, N, 16), jnp.float32),
    mesh=_mesh,
    scratch_types=[
        pltpu.VMEM((2, BD), jnp.int32),
        pltpu.VMEM((BD, 16), jnp.float32),
        pltpu.VMEM((ZR, 16), jnp.float32),
        pltpu.VMEM_SHARED((ACCN, 16), jnp.float32),
        pltpu.SemaphoreType.DMA,
        pltpu.SemaphoreType.DMA,
        pltpu.SemaphoreType.DMA,
    ],
)
def _deg_kernel(dst_hbm, out_hbm, didx_v, ones_v, zrows_v, acc_sh,
                si0, si1, semz):
    c = lax.axis_index("c")
    s = lax.axis_index("s")
    wid = c * NS + s
    ebase = wid * BPTD * BD

    pltpu.async_copy(dst_hbm.at[pl.ds(ebase, BD)], didx_v.at[0], si0)

    def ofill(r, carry):
        ones_v[r] = jnp.ones((16,), jnp.float32)
        return carry
    lax.fori_loop(0, BD, ofill, 0)

    _zero_shared(acc_sh, zrows_v, s, 1, semz)
    plsc.subcore_barrier()

    def pair(k, carry):
        o1 = ebase + (2 * k + 1) * BD
        pltpu.async_copy(dst_hbm.at[pl.ds(o1, BD)], didx_v.at[1], si1)
        pltpu.make_async_copy(
            dst_hbm.at[pl.ds(ebase, BD)], didx_v.at[0], si0).wait()
        pltpu.sync_copy(ones_v, acc_sh.at[didx_v.at[0]], add=True)

        @pl.when(k < DPAIRS - 1)
        def _():
            o2 = ebase + (2 * k + 2) * BD
            pltpu.async_copy(dst_hbm.at[pl.ds(o2, BD)], didx_v.at[0], si0)
        pltpu.make_async_copy(
            dst_hbm.at[pl.ds(o1, BD)], didx_v.at[1], si1).wait()
        pltpu.sync_copy(ones_v, acc_sh.at[didx_v.at[1]], add=True)
        return carry
    lax.fori_loop(0, DPAIRS, pair, 0)

    plsc.subcore_barrier()
    _copy_out(acc_sh, out_hbm, c, s)


def _copy_out(acc_sh, out_hbm, c, s):
    pltpu.sync_copy(acc_sh.at[pl.ds(s * RPT, RPT)],
                    out_hbm.at[c, pl.ds(s * RPT, RPT)])

    @pl.when(s == NS - 1)
    def _():
        pltpu.sync_copy(acc_sh.at[pl.ds(NS * RPT, TAIL)],
                        out_hbm.at[c, pl.ds(NS * RPT, TAIL)])


@functools.partial(
    pl.kernel,
    out_type=jax.ShapeDtypeStruct((NC, N, D), jnp.float32),
    mesh=_mesh,
    scratch_types=[
        pltpu.VMEM((2, B), jnp.int32),
        pltpu.VMEM((2, B), jnp.int32),
        pltpu.VMEM((B, D), jnp.float32),
        pltpu.VMEM((B, D), jnp.float32),
        pltpu.VMEM((ZR, D), jnp.float32),
        pltpu.VMEM_SHARED((ACCN, D), jnp.float32),
        pltpu.SemaphoreType.DMA,
        pltpu.SemaphoreType.DMA,
        pltpu.SemaphoreType.DMA,
    ],
)
def _agg_kernel(hp_hbm, src_hbm, dst_hbm, out_hbm,
                sidx_v, didx_v, r0_v, r1_v, zrows_v, acc_sh,
                sem0, sem1, semz):
    c = lax.axis_index("c")
    s = lax.axis_index("s")
    wid = c * NS + s
    ebase = wid * BPT * B

    pltpu.sync_copy(src_hbm.at[pl.ds(ebase, B)], sidx_v.at[0])
    pltpu.sync_copy(dst_hbm.at[pl.ds(ebase, B)], didx_v.at[0])
    _zero_shared(acc_sh, zrows_v, s, D // 16, semz)
    plsc.subcore_barrier()
    pltpu.async_copy(hp_hbm.at[sidx_v.at[0]], r0_v, sem0)

    def pair(k, carry):
        o1 = ebase + (2 * k + 1) * B
        pltpu.sync_copy(src_hbm.at[pl.ds(o1, B)], sidx_v.at[1])
        pltpu.sync_copy(dst_hbm.at[pl.ds(o1, B)], didx_v.at[1])
        pltpu.async_copy(hp_hbm.at[sidx_v.at[1]], r1_v, sem1)
        pltpu.make_async_copy(hp_hbm.at[sidx_v.at[0]], r0_v, sem0).wait()
        pltpu.sync_copy(r0_v, acc_sh.at[didx_v.at[0]], add=True)
        o2 = ebase + (2 * k + 2) * B
        pltpu.sync_copy(src_hbm.at[pl.ds(o2, B)], sidx_v.at[0])
        pltpu.sync_copy(dst_hbm.at[pl.ds(o2, B)], didx_v.at[0])
        pltpu.async_copy(hp_hbm.at[sidx_v.at[0]], r0_v, sem0)
        pltpu.make_async_copy(hp_hbm.at[sidx_v.at[1]], r1_v, sem1).wait()
        pltpu.sync_copy(r1_v, acc_sh.at[didx_v.at[1]], add=True)
        return carry
    lax.fori_loop(0, PAIRS, pair, 0)

    pltpu.make_async_copy(hp_hbm.at[sidx_v.at[0]], r0_v, sem0).wait()
    pltpu.sync_copy(r0_v, acc_sh.at[didx_v.at[0]], add=True)

    plsc.subcore_barrier()
    _copy_out(acc_sh, out_hbm, c, s)


ROWS_TC = 1000
GRID_TC = N // ROWS_TC


def _dis_block(p_ref):
    p0 = p_ref[0, :, 0:1]
    p1 = p_ref[1, :, 0:1]
    return lax.rsqrt(p0 + p1 + 1.0)


def _tc1_body(x_ref, w_ref, p_ref, o_ref):
    dis = _dis_block(p_ref)
    o_ref[...] = jnp.dot(x_ref[...], w_ref[...],
                         preferred_element_type=jnp.float32) * dis


def _tc2_body(a_ref, hp_ref, p_ref, b_ref, w_ref, o_ref):
    dis = _dis_block(p_ref)
    h = jnp.maximum((a_ref[0] + a_ref[1] + hp_ref[...]) * dis + b_ref[...],
                    0.0)
    o_ref[...] = jnp.dot(h, w_ref[...],
                         preferred_element_type=jnp.float32) * dis


def _tc3_body(a_ref, hp_ref, p_ref, b_ref, o_ref):
    dis = _dis_block(p_ref)
    o_ref[...] = (a_ref[0] + a_ref[1] + hp_ref[...]) * dis + b_ref[...]


_row_spec = pl.BlockSpec((ROWS_TC, D), lambda i: (i, 0))
_acc_spec = pl.BlockSpec((NC, ROWS_TC, D), lambda i: (0, i, 0))
_p_spec = pl.BlockSpec((NC, ROWS_TC, 16), lambda i: (0, i, 0))
_w_spec = pl.BlockSpec((D, D), lambda i: (0, 0))
_b_spec = pl.BlockSpec((1, D), lambda i: (0, 0))
_out_sds = jax.ShapeDtypeStruct((N, D), jnp.float32)

_tc1 = pl.pallas_call(
    _tc1_body, grid=(GRID_TC,),
    in_specs=[_row_spec, _w_spec, _p_spec],
    out_specs=_row_spec, out_shape=_out_sds)

_tc2 = pl.pallas_call(
    _tc2_body, grid=(GRID_TC,),
    in_specs=[_acc_spec, _row_spec, _p_spec, _b_spec, _w_spec],
    out_specs=_row_spec, out_shape=_out_sds)

_tc3 = pl.pallas_call(
    _tc3_body, grid=(GRID_TC,),
    in_specs=[_acc_spec, _row_spec, _p_spec, _b_spec],
    out_specs=_row_spec, out_shape=_out_sds)


def kernel(x, edge_index, W1, b1, W2, b2):
    src = edge_index[0].astype(jnp.int32)
    dst = edge_index[1].astype(jnp.int32)
    ept = E // NW
    padd = BPTD * BD - ept
    dst32 = dst.reshape(NW, ept)
    dummy_rows = (DUMMY + (jnp.arange(NW, dtype=jnp.int32) % 8))[:, None]
    dstp = jnp.concatenate(
        [dst32, jnp.broadcast_to(dummy_rows, (NW, padd))],
        axis=1).reshape(-1)
    b1r = b1.reshape(1, D)
    b2r = b2.reshape(1, D)

    degp = _deg_kernel(dstp)
    h1p = _tc1(x, W1, degp)
    a1 = _agg_kernel(h1p, src, dst)
    h2p = _tc2(a1, h1p, degp, b1r, W2)
    a2 = _agg_kernel(h2p, src, dst)
    out = _tc3(a2, h2p, degp, b2r)
    return out
